# Initial kernel scaffold; baseline (speedup 1.0000x reference)
import jax
import jax.numpy as jnp
from jax.experimental import pallas as pl


def kernel(token_ids, base_table, freq_bands, freq_table):
    raise NotImplementedError("write your pallas kernel here")


# speedup vs baseline: 1.1213x; 1.1213x over previous
import functools

import jax
import jax.numpy as jnp
from jax import lax
from jax.experimental import pallas as pl
from jax.experimental.pallas import tpu as pltpu
from jax.experimental.pallas import tpu_sc as plsc

D2 = 32
F = 8
L = 16
NC = 2
NS = 16
NW = NC * NS
CHUNK = 1024
SUB = 128
NSUB = CHUNK // SUB


def _emb_body(tid_hbm, base_hbm, freq_hbm, bands_hbm, out_hbm,
              idx_v, base_v, flog_v, out_v, bands_v,
              sem_b, sem_f):
    n_tokens = tid_hbm.shape[0] * tid_hbm.shape[1]
    tpw = n_tokens // NW
    n_chunks = tpw // CHUNK
    ngrp = CHUNK // L

    cid = lax.axis_index("c")
    sid = lax.axis_index("s")
    wid = sid * NC + cid

    pltpu.sync_copy(bands_hbm, bands_v)
    band_lo = [bands_v[f, pl.ds(0, L)] for f in range(F)]
    band_hi = [bands_v[f, pl.ds(L, L)] for f in range(F)]

    def chunk_body(c, _):
        tok0 = pl.multiple_of(wid * tpw + c * CHUNK, CHUNK)
        row0 = pl.multiple_of(tok0 // SUB, NSUB)

        pltpu.sync_copy(tid_hbm.at[pl.ds(row0, NSUB)], idx_v)
        cps_b = [pltpu.async_copy(base_hbm.at[idx_v.at[j]],
                                  base_v.at[pl.ds(j * SUB, SUB)], sem_b)
                 for j in range(NSUB)]
        cps_f = [pltpu.async_copy(freq_hbm.at[idx_v.at[j]],
                                  flog_v.at[pl.ds(j * SUB, SUB)], sem_f)
                 for j in range(NSUB)]
        for cp in cps_f:
            cp.wait()
        for cp in cps_b:
            cp.wait()

        def group_body(g, _):
            rows = g * L + lax.iota(jnp.int32, L)
            ls = [plsc.load_gather(flog_v, [rows, jnp.full((L,), f, jnp.int32)])
                  for f in range(F)]
            m01 = jnp.maximum(ls[0], ls[1])
            m23 = jnp.maximum(ls[2], ls[3])
            m45 = jnp.maximum(ls[4], ls[5])
            m67 = jnp.maximum(ls[6], ls[7])
            m = jnp.maximum(jnp.maximum(m01, m23), jnp.maximum(m45, m67))
            es = [jnp.exp(l - m) for l in ls]
            s = (((es[0] + es[1]) + (es[2] + es[3]))
                 + ((es[4] + es[5]) + (es[6] + es[7])))
            r = 1.0 / s
            ws = [e * r for e in es]

            for t in range(L):
                tok = g * L + t
                w0 = ws[0][t]
                acc_lo = w0 * band_lo[0]
                acc_hi = w0 * band_hi[0]
                for f in range(1, F):
                    wf = ws[f][t]
                    acc_lo = acc_lo + wf * band_lo[f]
                    acc_hi = acc_hi + wf * band_hi[f]
                out_v[tok, pl.ds(0, L)] = base_v[tok, pl.ds(0, L)]
                out_v[tok, pl.ds(L, L)] = base_v[tok, pl.ds(L, L)]
                out_v[tok, pl.ds(2 * L, L)] = acc_lo
                out_v[tok, pl.ds(3 * L, L)] = acc_hi
            return ()

        lax.fori_loop(0, ngrp, group_body, ())
        pltpu.sync_copy(out_v, out_hbm.at[pl.ds(tok0, CHUNK)])
        return ()

    lax.fori_loop(0, n_chunks, chunk_body, ())


@functools.partial(jax.jit, static_argnums=())
def _emb_call(ids2d, base_table, freq_table, freq_bands):
    n_tokens = ids2d.shape[0] * ids2d.shape[1]
    kern = functools.partial(
        pl.kernel,
        out_type=jax.ShapeDtypeStruct((n_tokens, 2 * D2), jnp.float32),
        mesh=plsc.VectorSubcoreMesh(core_axis_name="c", subcore_axis_name="s"),
        scratch_types=[
            pltpu.VMEM((NSUB, SUB), jnp.int32),
            pltpu.VMEM((CHUNK, D2), jnp.float32),
            pltpu.VMEM((CHUNK, F), jnp.float32),
            pltpu.VMEM((CHUNK, 2 * D2), jnp.float32),
            pltpu.VMEM((F, D2), jnp.float32),
            pltpu.SemaphoreType.DMA,
            pltpu.SemaphoreType.DMA,
        ],
        compiler_params=pltpu.CompilerParams(
            needs_layout_passes=False, use_tc_tiling_on_sc=False),
    )(_emb_body)
    return kern(ids2d, base_table, freq_table, freq_bands)


def kernel(token_ids, base_table, freq_bands, freq_table):
    b, s = token_ids.shape
    ids2d = token_ids.astype(jnp.int32).reshape((b * s) // SUB, SUB)
    out = _emb_call(ids2d, base_table, freq_table, freq_bands)
    return out.reshape(b, s, 2 * D2)
